# baseline (speedup 1.0000x reference)
import functools

import numpy as np
import jax
import jax.numpy as jnp
from jax import lax
from jax.experimental import pallas as pl
from jax.experimental.pallas import tpu as pltpu
from jax.experimental.pallas import tpu_sc as plsc

_N_LEVELS = 16
_F = 4
_T = 2 ** 19
_BASE_RES = 16
_BBOX = 4.0
_P1 = int(np.uint32(2654435761).astype(np.int32))
_P2 = 805459861

_LEVELS = []
for _l in range(_N_LEVELS):
    _scale = float(np.exp2(_l * np.log2(1.5)) * _BASE_RES) - 1.0
    _res = int(np.ceil(_scale)) + 1
    _LEVELS.append((_scale, _res, (_res ** 3) <= _T))

_NC, _NS = 2, 16
_NW = _NC * _NS
_C = 64
_SUB = _C // 16
_NIDX = _C * _N_LEVELS * 8


def _sc_body(pts_hbm, table_hbm, feats_hbm, pts_v, idx_v, par_v, rows_v,
             feats_v, sem):
    n_pts = pts_hbm.shape[0]
    np_per_w = n_pts // _NW
    n_chunks = np_per_w // _C

    wid = lax.axis_index("s") * _NC + lax.axis_index("c")
    base = wid * np_per_w

    iota = lax.iota(jnp.int32, 16)
    col0 = jnp.zeros((16,), jnp.int32)
    col1 = jnp.full((16,), 1, jnp.int32)
    col2 = jnp.full((16,), 2, jnp.int32)

    def load_xyz(s):
        row = s * 16 + iota
        x = plsc.load_gather(pts_v, [row, col0])
        y = plsc.load_gather(pts_v, [row, col1])
        z = plsc.load_gather(pts_v, [row, col2])
        x = x * (1.0 / _BBOX) + 0.5
        y = y * (1.0 / _BBOX) + 0.5
        z = z * (1.0 / _BBOX) + 0.5
        return x, y, z

    def level_pos(x, y, z, scale):
        px = x * scale + 0.5
        py = y * scale + 0.5
        pz = z * scale + 0.5
        gx = px.astype(jnp.int32)
        gy = py.astype(jnp.int32)
        gz = pz.astype(jnp.int32)
        return px, py, pz, gx, gy, gz

    def corner_idx(l, gx, gy, gz):
        scale, res, dense = _LEVELS[l]
        out = []
        for c in range(8):
            cx = gx + (c & 1) if (c & 1) else gx
            cy = gy + ((c >> 1) & 1) if ((c >> 1) & 1) else gy
            cz = gz + ((c >> 2) & 1) if ((c >> 2) & 1) else gz
            if dense:
                idx = cx + cy * res + cz * (res * res)
            else:
                idx = (cx ^ (cy * _P1) ^ (cz * _P2)) & (_T - 1)
            out.append(idx + l * _T)
        return out

    def chunk_body(g, _):
        cbase = base + g * _C
        pltpu.sync_copy(pts_hbm.at[pl.ds(cbase, _C)], pts_v)

        def sub_idx(s, _):
            x, y, z = load_xyz(s)
            for l in range(_N_LEVELS):
                scale, res, dense = _LEVELS[l]
                _, _, _, gx, gy, gz = level_pos(x, y, z, scale)
                idxs = corner_idx(l, gx, gy, gz)
                for c in range(8):
                    off = s * (_N_LEVELS * 8 * 16) + (l * 8 + c) * 16
                    e = idxs[c]
                    idx_v[pl.ds(off, 16)] = lax.shift_right_logical(e, 1)
                    par_v[pl.ds(off, 16)] = (e & 1) * _F
            return 0
        lax.fori_loop(0, _SUB, sub_idx, 0)

        pltpu.async_copy(table_hbm.at[idx_v], rows_v, sem).wait()

        def sub_interp(s, _):
            x, y, z = load_xyz(s)
            for l in range(_N_LEVELS):
                scale, res, dense = _LEVELS[l]
                px, py, pz, gx, gy, gz = level_pos(x, y, z, scale)
                wx = px - gx.astype(jnp.float32)
                wy = py - gy.astype(jnp.float32)
                wz = pz - gz.astype(jnp.float32)
                ux = 1.0 - wx
                uy = 1.0 - wy
                uz = 1.0 - wz
                a = ux * uy
                b = wx * uy
                cc = ux * wy
                d = wx * wy
                w8 = [a * uz, b * uz, cc * uz, d * uz,
                      a * wz, b * wz, cc * wz, d * wz]
                acc = [None] * _F
                for c in range(8):
                    off = s * (_N_LEVELS * 8 * 16) + (l * 8 + c) * 16
                    rbase = off + iota
                    par4 = par_v[pl.ds(off, 16)]
                    for f in range(_F):
                        v = plsc.load_gather(rows_v, [rbase, par4 + f])
                        t = w8[c] * v
                        acc[f] = t if acc[f] is None else acc[f] + t
                row = s * 16 + iota
                for f in range(_F):
                    plsc.store_scatter(
                        feats_v, [row, jnp.full((16,), l * _F + f, jnp.int32)],
                        acc[f])
            return 0
        lax.fori_loop(0, _SUB, sub_interp, 0)

        pltpu.sync_copy(feats_v, feats_hbm.at[pl.ds(cbase, _C)])
        return 0

    lax.fori_loop(0, n_chunks, chunk_body, 0)


def _sc_features(pts, table2):
    n_pts = pts.shape[0]
    mesh = plsc.VectorSubcoreMesh(core_axis_name="c", subcore_axis_name="s")
    return pl.kernel(
        _sc_body,
        out_type=jax.ShapeDtypeStruct((n_pts, _N_LEVELS * _F), jnp.float32),
        mesh=mesh,
        compiler_params=pltpu.CompilerParams(
            needs_layout_passes=False, use_tc_tiling_on_sc=False),
        scratch_types=[
            pltpu.VMEM((_C, 3), jnp.float32),
            pltpu.VMEM((_NIDX,), jnp.int32),
            pltpu.VMEM((_NIDX,), jnp.int32),
            pltpu.VMEM((_NIDX, 2 * _F), jnp.float32),
            pltpu.VMEM((_C, _N_LEVELS * _F), jnp.float32),
            pltpu.SemaphoreType.DMA,
        ],
    )(pts, table2)


_SH_C = (0.28209479177387814, -0.48860251190291987, 0.48860251190291987,
         -0.48860251190291987, 1.0925484305920792, -1.0925484305920792,
         0.94617469575755997, -0.31539156525251999, -1.0925484305920792,
         0.54627421529603959, 0.59004358992664352, 2.8906114426405538,
         0.45704579946446572, 0.3731763325901154, 1.4453057213202769)


def _mlp_body(feats_ref, vd_ref, ws1, ws2, wc1g, wc1s, wc2, wc3,
              sig_ref, rgb_ref):
    f = feats_ref[...]
    h1 = jnp.maximum(jnp.dot(f, ws1[...], preferred_element_type=jnp.float32),
                     0.0)
    so = jnp.dot(h1, ws2[...], preferred_element_type=jnp.float32)
    sig_ref[...] = jnp.maximum(so[:, 0:1], 0.0)

    d = vd_ref[...] * 2.0 - 1.0
    x = d[:, 0:1]
    y = d[:, 1:2]
    z = d[:, 2:3]
    xy = x * y
    xz = x * z
    yz = y * z
    x2 = x * x
    y2 = y * y
    z2 = z * z
    sh = jnp.concatenate([
        _SH_C[0] * jnp.ones_like(x),
        _SH_C[1] * y,
        _SH_C[2] * z,
        _SH_C[3] * x,
        _SH_C[4] * xy,
        _SH_C[5] * yz,
        0.94617469575755997 * z2 - 0.31539156525251999,
        _SH_C[8] * xz,
        _SH_C[9] * (x2 - y2),
        _SH_C[10] * y * (-3.0 * x2 + y2),
        _SH_C[11] * xy * z,
        _SH_C[12] * y * (1.0 - 5.0 * z2),
        _SH_C[13] * z * (5.0 * z2 - 3.0),
        _SH_C[12] * x * (1.0 - 5.0 * z2),
        _SH_C[14] * z * (x2 - y2),
        _SH_C[10] * x * (-x2 + 3.0 * y2),
    ], axis=1)

    pre = (jnp.dot(so, wc1g[...], preferred_element_type=jnp.float32)
           + jnp.dot(sh, wc1s[...], preferred_element_type=jnp.float32))
    h = jnp.maximum(pre, 0.0)
    h = jnp.maximum(jnp.dot(h, wc2[...], preferred_element_type=jnp.float32),
                    0.0)
    rgb_ref[...] = jax.nn.sigmoid(
        jnp.dot(h, wc3[...], preferred_element_type=jnp.float32))


def _tc_mlp(feats, viewdirs, W_s1, W_s2, W_c1, W_c2, W_c3):
    n = feats.shape[0]
    B = 2048
    grid = n // B
    wc1g = jnp.concatenate([jnp.zeros((1, 64), jnp.float32), W_c1[:15]], axis=0)
    wc1s = W_c1[15:]
    rep = lambda shape: pl.BlockSpec(shape, lambda i: (0, 0))
    sig, rgb = pl.pallas_call(
        _mlp_body,
        grid=(grid,),
        in_specs=[
            pl.BlockSpec((B, 64), lambda i: (i, 0)),
            pl.BlockSpec((B, 3), lambda i: (i, 0)),
            rep((64, 64)), rep((64, 16)), rep((16, 64)), rep((16, 64)),
            rep((64, 64)), rep((64, 3)),
        ],
        out_specs=[
            pl.BlockSpec((B, 1), lambda i: (i, 0)),
            pl.BlockSpec((B, 3), lambda i: (i, 0)),
        ],
        out_shape=[
            jax.ShapeDtypeStruct((n, 1), jnp.float32),
            jax.ShapeDtypeStruct((n, 3), jnp.float32),
        ],
        compiler_params=pltpu.CompilerParams(
            dimension_semantics=("parallel",)),
    )(feats, viewdirs, W_s1, W_s2, wc1g, wc1s, W_c2, W_c3)
    return sig, rgb


def kernel(pts, viewdirs, table, W_s1, W_s2, W_c1, W_c2, W_c3):
    table2 = table.reshape(_N_LEVELS * _T // 2, 2 * _F)
    feats = _sc_features(pts, table2)
    sig, rgb = _tc_mlp(feats, viewdirs, W_s1, W_s2, W_c1, W_c2, W_c3)
    return (sig.reshape(pts.shape[0]), rgb)


# speedup vs baseline: 1.3571x; 1.3571x over previous
import functools

import numpy as np
import jax
import jax.numpy as jnp
from jax import lax
from jax.experimental import pallas as pl
from jax.experimental.pallas import tpu as pltpu
from jax.experimental.pallas import tpu_sc as plsc

_N_LEVELS = 16
_F = 4
_T = 2 ** 19
_BASE_RES = 16
_BBOX = 4.0
_P1 = int(np.uint32(2654435761).astype(np.int32))
_P2 = 805459861

_LEVELS = []
for _l in range(_N_LEVELS):
    _scale = float(np.exp2(_l * np.log2(1.5)) * _BASE_RES) - 1.0
    _res = int(np.ceil(_scale)) + 1
    _LEVELS.append((_scale, _res, (_res ** 3) <= _T))

_NC, _NS = 2, 16
_NW = _NC * _NS
_C = 64
_SUB = _C // 16
_NIDX = _C * _N_LEVELS * 8


def _sc_body(pts_hbm, table_hbm, feats_hbm, pts_v, idx_v, rows_v,
             feats_v, sem):
    n_pts = pts_hbm.shape[0]
    np_per_w = n_pts // _NW
    n_chunks = np_per_w // _C

    wid = lax.axis_index("s") * _NC + lax.axis_index("c")
    base = wid * np_per_w

    iota = lax.iota(jnp.int32, 16)
    col0 = jnp.zeros((16,), jnp.int32)
    col1 = jnp.full((16,), 1, jnp.int32)
    col2 = jnp.full((16,), 2, jnp.int32)

    def load_xyz(s):
        row = s * 16 + iota
        x = plsc.load_gather(pts_v, [row, col0])
        y = plsc.load_gather(pts_v, [row, col1])
        z = plsc.load_gather(pts_v, [row, col2])
        x = x * (1.0 / _BBOX) + 0.5
        y = y * (1.0 / _BBOX) + 0.5
        z = z * (1.0 / _BBOX) + 0.5
        return x, y, z

    def level_pos(x, y, z, scale):
        px = x * scale + 0.5
        py = y * scale + 0.5
        pz = z * scale + 0.5
        gx = px.astype(jnp.int32)
        gy = py.astype(jnp.int32)
        gz = pz.astype(jnp.int32)
        return px, py, pz, gx, gy, gz

    def corner_idx(l, gx, gy, gz):
        scale, res, dense = _LEVELS[l]
        out = []
        for c in range(8):
            cx = gx + (c & 1) if (c & 1) else gx
            cy = gy + ((c >> 1) & 1) if ((c >> 1) & 1) else gy
            cz = gz + ((c >> 2) & 1) if ((c >> 2) & 1) else gz
            if dense:
                idx = cx + cy * res + cz * (res * res)
            else:
                idx = (cx ^ (cy * _P1) ^ (cz * _P2)) & (_T - 1)
            base = ((idx >> 7) << 9) + (idx & 127) + l * (_T * _F)
            out.append(base)
        return out

    def chunk_body(g, _):
        cbase = base + g * _C
        pltpu.sync_copy(pts_hbm.at[pl.ds(cbase, _C)], pts_v)

        def sub_idx(s, _):
            x, y, z = load_xyz(s)
            for l in range(_N_LEVELS):
                scale, res, dense = _LEVELS[l]
                _, _, _, gx, gy, gz = level_pos(x, y, z, scale)
                idxs = corner_idx(l, gx, gy, gz)
                for c in range(8):
                    off = (s * (_N_LEVELS * 8 * 16) + (l * 8 + c) * 16) * _F
                    base = idxs[c]
                    for f in range(_F):
                        idx_v[pl.ds(off + f * 16, 16)] = base + f * 128
            return 0
        lax.fori_loop(0, _SUB, sub_idx, 0)

        pltpu.async_copy(table_hbm.at[idx_v], rows_v, sem).wait()

        def sub_interp(s, _):
            x, y, z = load_xyz(s)
            for l in range(_N_LEVELS):
                scale, res, dense = _LEVELS[l]
                px, py, pz, gx, gy, gz = level_pos(x, y, z, scale)
                wx = px - gx.astype(jnp.float32)
                wy = py - gy.astype(jnp.float32)
                wz = pz - gz.astype(jnp.float32)
                ux = 1.0 - wx
                uy = 1.0 - wy
                uz = 1.0 - wz
                a = ux * uy
                b = wx * uy
                cc = ux * wy
                d = wx * wy
                w8 = [a * uz, b * uz, cc * uz, d * uz,
                      a * wz, b * wz, cc * wz, d * wz]
                acc = [None] * _F
                for c in range(8):
                    off = (s * (_N_LEVELS * 8 * 16) + (l * 8 + c) * 16) * _F
                    for f in range(_F):
                        v = rows_v[pl.ds(off + f * 16, 16)]
                        t = w8[c] * v
                        acc[f] = t if acc[f] is None else acc[f] + t
                row = s * 16 + iota
                for f in range(_F):
                    plsc.store_scatter(
                        feats_v, [row, jnp.full((16,), l * _F + f, jnp.int32)],
                        acc[f])
            return 0
        lax.fori_loop(0, _SUB, sub_interp, 0)

        pltpu.sync_copy(feats_v, feats_hbm.at[pl.ds(cbase, _C)])
        return 0

    lax.fori_loop(0, n_chunks, chunk_body, 0)


def _sc_features(pts, table2):
    n_pts = pts.shape[0]
    mesh = plsc.VectorSubcoreMesh(core_axis_name="c", subcore_axis_name="s")
    return pl.kernel(
        _sc_body,
        out_type=jax.ShapeDtypeStruct((n_pts, _N_LEVELS * _F), jnp.float32),
        mesh=mesh,
        compiler_params=pltpu.CompilerParams(
            needs_layout_passes=False, use_tc_tiling_on_sc=False),
        scratch_types=[
            pltpu.VMEM((_C, 3), jnp.float32),
            pltpu.VMEM((_NIDX * _F,), jnp.int32),
            pltpu.VMEM((_NIDX * _F,), jnp.float32),
            pltpu.VMEM((_C, _N_LEVELS * _F), jnp.float32),
            pltpu.SemaphoreType.DMA,
        ],
    )(pts, table2)


_SH_C = (0.28209479177387814, -0.48860251190291987, 0.48860251190291987,
         -0.48860251190291987, 1.0925484305920792, -1.0925484305920792,
         0.94617469575755997, -0.31539156525251999, -1.0925484305920792,
         0.54627421529603959, 0.59004358992664352, 2.8906114426405538,
         0.45704579946446572, 0.3731763325901154, 1.4453057213202769)


def _mlp_body(feats_ref, vd_ref, ws1, ws2, wc1g, wc1s, wc2, wc3,
              sig_ref, rgb_ref):
    f = feats_ref[...]
    h1 = jnp.maximum(jnp.dot(f, ws1[...], preferred_element_type=jnp.float32),
                     0.0)
    so = jnp.dot(h1, ws2[...], preferred_element_type=jnp.float32)
    sig_ref[...] = jnp.maximum(so[:, 0:1], 0.0)

    d = vd_ref[...] * 2.0 - 1.0
    x = d[:, 0:1]
    y = d[:, 1:2]
    z = d[:, 2:3]
    xy = x * y
    xz = x * z
    yz = y * z
    x2 = x * x
    y2 = y * y
    z2 = z * z
    sh = jnp.concatenate([
        _SH_C[0] * jnp.ones_like(x),
        _SH_C[1] * y,
        _SH_C[2] * z,
        _SH_C[3] * x,
        _SH_C[4] * xy,
        _SH_C[5] * yz,
        0.94617469575755997 * z2 - 0.31539156525251999,
        _SH_C[8] * xz,
        _SH_C[9] * (x2 - y2),
        _SH_C[10] * y * (-3.0 * x2 + y2),
        _SH_C[11] * xy * z,
        _SH_C[12] * y * (1.0 - 5.0 * z2),
        _SH_C[13] * z * (5.0 * z2 - 3.0),
        _SH_C[12] * x * (1.0 - 5.0 * z2),
        _SH_C[14] * z * (x2 - y2),
        _SH_C[10] * x * (-x2 + 3.0 * y2),
    ], axis=1)

    pre = (jnp.dot(so, wc1g[...], preferred_element_type=jnp.float32)
           + jnp.dot(sh, wc1s[...], preferred_element_type=jnp.float32))
    h = jnp.maximum(pre, 0.0)
    h = jnp.maximum(jnp.dot(h, wc2[...], preferred_element_type=jnp.float32),
                    0.0)
    rgb_ref[...] = jax.nn.sigmoid(
        jnp.dot(h, wc3[...], preferred_element_type=jnp.float32))


def _tc_mlp(feats, viewdirs, W_s1, W_s2, W_c1, W_c2, W_c3):
    n = feats.shape[0]
    B = 2048
    grid = n // B
    wc1g = jnp.concatenate([jnp.zeros((1, 64), jnp.float32), W_c1[:15]], axis=0)
    wc1s = W_c1[15:]
    rep = lambda shape: pl.BlockSpec(shape, lambda i: (0, 0))
    sig, rgb = pl.pallas_call(
        _mlp_body,
        grid=(grid,),
        in_specs=[
            pl.BlockSpec((B, 64), lambda i: (i, 0)),
            pl.BlockSpec((B, 3), lambda i: (i, 0)),
            rep((64, 64)), rep((64, 16)), rep((16, 64)), rep((16, 64)),
            rep((64, 64)), rep((64, 3)),
        ],
        out_specs=[
            pl.BlockSpec((B, 1), lambda i: (i, 0)),
            pl.BlockSpec((B, 3), lambda i: (i, 0)),
        ],
        out_shape=[
            jax.ShapeDtypeStruct((n, 1), jnp.float32),
            jax.ShapeDtypeStruct((n, 3), jnp.float32),
        ],
        compiler_params=pltpu.CompilerParams(
            dimension_semantics=("parallel",)),
    )(feats, viewdirs, W_s1, W_s2, wc1g, wc1s, W_c2, W_c3)
    return sig, rgb


def kernel(pts, viewdirs, table, W_s1, W_s2, W_c1, W_c2, W_c3):
    table1d = table.reshape(
        _N_LEVELS, _T // 128, 128, _F).transpose(0, 1, 3, 2).reshape(-1)
    feats = _sc_features(pts, table1d)
    sig, rgb = _tc_mlp(feats, viewdirs, W_s1, W_s2, W_c1, W_c2, W_c3)
    return (sig.reshape(pts.shape[0]), rgb)


# speedup vs baseline: 2.3465x; 1.7291x over previous
import functools

import numpy as np
import jax
import jax.numpy as jnp
from jax import lax
from jax.experimental import pallas as pl
from jax.experimental.pallas import tpu as pltpu
from jax.experimental.pallas import tpu_sc as plsc

_N_LEVELS = 16
_F = 4
_T = 2 ** 19
_BASE_RES = 16
_BBOX = 4.0
_P1 = int(np.uint32(2654435761).astype(np.int32))
_P2 = 805459861

_LEVELS = []
for _l in range(_N_LEVELS):
    _scale = float(np.exp2(_l * np.log2(1.5)) * _BASE_RES) - 1.0
    _res = int(np.ceil(_scale)) + 1
    _LEVELS.append((_scale, _res, (_res ** 3) <= _T))

_NC, _NS = 2, 16
_NW = _NC * _NS
_C = 64
_SUB = _C // 16
_NIDX = _C * _N_LEVELS * 8


def _sc_body(pts_hbm, table_hbm, feats_hbm, pts_v, idx_v, par_v, rows_v,
             feats_v, sem):
    n_pts = pts_hbm.shape[0]
    np_per_w = n_pts // _NW
    n_chunks = np_per_w // _C

    wid = lax.axis_index("s") * _NC + lax.axis_index("c")
    base = wid * np_per_w

    iota = lax.iota(jnp.int32, 16)
    col0 = jnp.zeros((16,), jnp.int32)
    col1 = jnp.full((16,), 1, jnp.int32)
    col2 = jnp.full((16,), 2, jnp.int32)

    def load_xyz(s):
        row = s * 16 + iota
        x = plsc.load_gather(pts_v, [row, col0])
        y = plsc.load_gather(pts_v, [row, col1])
        z = plsc.load_gather(pts_v, [row, col2])
        x = x * (1.0 / _BBOX) + 0.5
        y = y * (1.0 / _BBOX) + 0.5
        z = z * (1.0 / _BBOX) + 0.5
        return x, y, z

    def level_pos(x, y, z, scale):
        px = x * scale + 0.5
        py = y * scale + 0.5
        pz = z * scale + 0.5
        gx = px.astype(jnp.int32)
        gy = py.astype(jnp.int32)
        gz = pz.astype(jnp.int32)
        return px, py, pz, gx, gy, gz

    def corner_idx(l, gx, gy, gz):
        scale, res, dense = _LEVELS[l]
        out = []
        for c in range(8):
            cx = gx + (c & 1) if (c & 1) else gx
            cy = gy + ((c >> 1) & 1) if ((c >> 1) & 1) else gy
            cz = gz + ((c >> 2) & 1) if ((c >> 2) & 1) else gz
            if dense:
                idx = cx + cy * res + cz * (res * res)
            else:
                idx = (cx ^ (cy * _P1) ^ (cz * _P2)) & (_T - 1)
            out.append(idx + l * _T)
        return out

    def chunk_body(g, _):
        cbase = base + g * _C
        pltpu.sync_copy(pts_hbm.at[pl.ds(cbase, _C)], pts_v)

        def sub_idx(s, _):
            x, y, z = load_xyz(s)
            for l in range(_N_LEVELS):
                scale, res, dense = _LEVELS[l]
                _, _, _, gx, gy, gz = level_pos(x, y, z, scale)
                idxs = corner_idx(l, gx, gy, gz)
                for c in range(8):
                    off = s * (_N_LEVELS * 8 * 16) + (l * 8 + c) * 16
                    e = idxs[c]
                    idx_v[pl.ds(off, 16)] = e >> 1
                    par_v[pl.ds(off, 16)] = (e & 1) * _F
            return 0
        lax.fori_loop(0, _SUB, sub_idx, 0)

        pltpu.async_copy(table_hbm.at[idx_v], rows_v, sem).wait()

        def sub_interp(s, _):
            x, y, z = load_xyz(s)
            for l in range(_N_LEVELS):
                scale, res, dense = _LEVELS[l]
                px, py, pz, gx, gy, gz = level_pos(x, y, z, scale)
                wx = px - gx.astype(jnp.float32)
                wy = py - gy.astype(jnp.float32)
                wz = pz - gz.astype(jnp.float32)
                ux = 1.0 - wx
                uy = 1.0 - wy
                uz = 1.0 - wz
                a = ux * uy
                b = wx * uy
                cc = ux * wy
                d = wx * wy
                w8 = [a * uz, b * uz, cc * uz, d * uz,
                      a * wz, b * wz, cc * wz, d * wz]
                acc = [None] * _F
                for c in range(8):
                    off = s * (_N_LEVELS * 8 * 16) + (l * 8 + c) * 16
                    rbase = off + iota
                    par4 = par_v[pl.ds(off, 16)]
                    for f in range(_F):
                        v = plsc.load_gather(rows_v, [rbase, par4 + f])
                        t = w8[c] * v
                        acc[f] = t if acc[f] is None else acc[f] + t
                row = s * 16 + iota
                for f in range(_F):
                    plsc.store_scatter(
                        feats_v, [row, jnp.full((16,), l * _F + f, jnp.int32)],
                        acc[f])
            return 0
        lax.fori_loop(0, _SUB, sub_interp, 0)

        pltpu.sync_copy(feats_v, feats_hbm.at[pl.ds(cbase, _C)])
        return 0

    lax.fori_loop(0, n_chunks, chunk_body, 0)


_RG = 64


def _relayout_body(src_hbm, dst_hbm, in_v, out_v):
    wid = lax.axis_index("s") * _NC + lax.axis_index("c")
    iota = lax.iota(jnp.int32, 16)
    iota4 = iota * 4
    n = src_hbm.shape[0]
    per_w = n // _NW
    chunk = _RG * 512
    n_chunks = per_w // chunk
    base_w = wid * per_w

    def chunk_body(t, _):
        base = base_w + t * chunk
        pltpu.sync_copy(src_hbm.at[pl.ds(base, chunk)], in_v)

        def g_body(g, _):
            g512 = g * 512
            for f in range(_F):
                for h in range(8):
                    v = in_v[pl.ds(g512 + f * 128 + h * 16, 16)]
                    plsc.store_scatter(out_v, [g512 + h * 64 + f + iota4], v)
            return 0
        lax.fori_loop(0, _RG, g_body, 0)
        pltpu.sync_copy(out_v, dst_hbm.at[pl.ds(base, chunk)])
        return 0
    lax.fori_loop(0, n_chunks, chunk_body, 0)


def _sc_relayout(table1d):
    mesh = plsc.VectorSubcoreMesh(core_axis_name="c", subcore_axis_name="s")
    out = pl.kernel(
        _relayout_body,
        out_type=jax.ShapeDtypeStruct(table1d.shape, jnp.float32),
        mesh=mesh,
        compiler_params=pltpu.CompilerParams(
            needs_layout_passes=False, use_tc_tiling_on_sc=False),
        scratch_types=[
            pltpu.VMEM((_RG * 512,), jnp.float32),
            pltpu.VMEM((_RG * 512,), jnp.float32),
        ],
    )(table1d)
    return out.reshape(_N_LEVELS * _T // 2, 2 * _F)


def _sc_features(pts, table2):
    n_pts = pts.shape[0]
    mesh = plsc.VectorSubcoreMesh(core_axis_name="c", subcore_axis_name="s")
    return pl.kernel(
        _sc_body,
        out_type=jax.ShapeDtypeStruct((n_pts, _N_LEVELS * _F), jnp.float32),
        mesh=mesh,
        compiler_params=pltpu.CompilerParams(
            needs_layout_passes=False, use_tc_tiling_on_sc=False),
        scratch_types=[
            pltpu.VMEM((_C, 3), jnp.float32),
            pltpu.VMEM((_NIDX,), jnp.int32),
            pltpu.VMEM((_NIDX,), jnp.int32),
            pltpu.VMEM((_NIDX, 2 * _F), jnp.float32),
            pltpu.VMEM((_C, _N_LEVELS * _F), jnp.float32),
            pltpu.SemaphoreType.DMA,
        ],
    )(pts, table2)


_SH_C = (0.28209479177387814, -0.48860251190291987, 0.48860251190291987,
         -0.48860251190291987, 1.0925484305920792, -1.0925484305920792,
         0.94617469575755997, -0.31539156525251999, -1.0925484305920792,
         0.54627421529603959, 0.59004358992664352, 2.8906114426405538,
         0.45704579946446572, 0.3731763325901154, 1.4453057213202769)


def _mlp_body(feats_ref, vd_ref, ws1, ws2, wc1g, wc1s, wc2, wc3,
              sig_ref, rgb_ref):
    f = feats_ref[...]
    h1 = jnp.maximum(jnp.dot(f, ws1[...], preferred_element_type=jnp.float32),
                     0.0)
    so = jnp.dot(h1, ws2[...], preferred_element_type=jnp.float32)
    sig_ref[...] = jnp.maximum(so[:, 0:1], 0.0)

    d = vd_ref[...] * 2.0 - 1.0
    x = d[:, 0:1]
    y = d[:, 1:2]
    z = d[:, 2:3]
    xy = x * y
    xz = x * z
    yz = y * z
    x2 = x * x
    y2 = y * y
    z2 = z * z
    sh = jnp.concatenate([
        _SH_C[0] * jnp.ones_like(x),
        _SH_C[1] * y,
        _SH_C[2] * z,
        _SH_C[3] * x,
        _SH_C[4] * xy,
        _SH_C[5] * yz,
        0.94617469575755997 * z2 - 0.31539156525251999,
        _SH_C[8] * xz,
        _SH_C[9] * (x2 - y2),
        _SH_C[10] * y * (-3.0 * x2 + y2),
        _SH_C[11] * xy * z,
        _SH_C[12] * y * (1.0 - 5.0 * z2),
        _SH_C[13] * z * (5.0 * z2 - 3.0),
        _SH_C[12] * x * (1.0 - 5.0 * z2),
        _SH_C[14] * z * (x2 - y2),
        _SH_C[10] * x * (-x2 + 3.0 * y2),
    ], axis=1)

    pre = (jnp.dot(so, wc1g[...], preferred_element_type=jnp.float32)
           + jnp.dot(sh, wc1s[...], preferred_element_type=jnp.float32))
    h = jnp.maximum(pre, 0.0)
    h = jnp.maximum(jnp.dot(h, wc2[...], preferred_element_type=jnp.float32),
                    0.0)
    rgb_ref[...] = jax.nn.sigmoid(
        jnp.dot(h, wc3[...], preferred_element_type=jnp.float32))


def _tc_mlp(feats, viewdirs, W_s1, W_s2, W_c1, W_c2, W_c3):
    n = feats.shape[0]
    B = 2048
    grid = n // B
    wc1g = jnp.concatenate([jnp.zeros((1, 64), jnp.float32), W_c1[:15]], axis=0)
    wc1s = W_c1[15:]
    rep = lambda shape: pl.BlockSpec(shape, lambda i: (0, 0))
    sig, rgb = pl.pallas_call(
        _mlp_body,
        grid=(grid,),
        in_specs=[
            pl.BlockSpec((B, 64), lambda i: (i, 0)),
            pl.BlockSpec((B, 3), lambda i: (i, 0)),
            rep((64, 64)), rep((64, 16)), rep((16, 64)), rep((16, 64)),
            rep((64, 64)), rep((64, 3)),
        ],
        out_specs=[
            pl.BlockSpec((B, 1), lambda i: (i, 0)),
            pl.BlockSpec((B, 3), lambda i: (i, 0)),
        ],
        out_shape=[
            jax.ShapeDtypeStruct((n, 1), jnp.float32),
            jax.ShapeDtypeStruct((n, 3), jnp.float32),
        ],
        compiler_params=pltpu.CompilerParams(
            dimension_semantics=("parallel",)),
    )(feats, viewdirs, W_s1, W_s2, wc1g, wc1s, W_c2, W_c3)
    return sig, rgb


def kernel(pts, viewdirs, table, W_s1, W_s2, W_c1, W_c2, W_c3):
    table1d = table.reshape(
        _N_LEVELS, _T // 128, 128, _F).transpose(0, 1, 3, 2).reshape(-1)
    table2 = _sc_relayout(table1d)
    feats = _sc_features(pts, table2)
    sig, rgb = _tc_mlp(feats, viewdirs, W_s1, W_s2, W_c1, W_c2, W_c3)
    return (sig.reshape(pts.shape[0]), rgb)


# speedup vs baseline: 3.3100x; 1.4106x over previous
import functools

import numpy as np
import jax
import jax.numpy as jnp
from jax import lax
from jax.experimental import pallas as pl
from jax.experimental.pallas import tpu as pltpu
from jax.experimental.pallas import tpu_sc as plsc

_N_LEVELS = 16
_F = 4
_T = 2 ** 19
_BASE_RES = 16
_BBOX = 4.0
_P1 = int(np.uint32(2654435761).astype(np.int32))
_P2 = 805459861

_LEVELS = []
for _l in range(_N_LEVELS):
    _scale = float(np.exp2(_l * np.log2(1.5)) * _BASE_RES) - 1.0
    _res = int(np.ceil(_scale)) + 1
    _LEVELS.append((_scale, _res, (_res ** 3) <= _T))

_NC, _NS = 2, 16
_NW = _NC * _NS
_C = 32
_SUB = _C // 16
_NIDX = _C * _N_LEVELS * 8


def _sc_body(pts_hbm, table_hbm, feats_hbm, pts_v, idx_v0, idx_v1,
             rows_v0, rows_v1, feats_v, sem0, sem1):
    n_pts = pts_hbm.shape[0]
    np_per_w = n_pts // _NW
    n_chunks = np_per_w // _C

    wid = lax.axis_index("s") * _NC + lax.axis_index("c")
    base = wid * np_per_w

    iota = lax.iota(jnp.int32, 16)
    col0 = jnp.zeros((16,), jnp.int32)
    col1 = jnp.full((16,), 1, jnp.int32)
    col2 = jnp.full((16,), 2, jnp.int32)
    idx_bufs = (idx_v0, idx_v1)
    rows_bufs = (rows_v0, rows_v1)
    sems = (sem0, sem1)

    def load_xyz(s, b):
        row = b * _C + s * 16 + iota
        x = plsc.load_gather(pts_v, [row, col0])
        y = plsc.load_gather(pts_v, [row, col1])
        z = plsc.load_gather(pts_v, [row, col2])
        x = x * (1.0 / _BBOX) + 0.5
        y = y * (1.0 / _BBOX) + 0.5
        z = z * (1.0 / _BBOX) + 0.5
        return x, y, z

    def level_pos(x, y, z, scale):
        px = x * scale + 0.5
        py = y * scale + 0.5
        pz = z * scale + 0.5
        gx = px.astype(jnp.int32)
        gy = py.astype(jnp.int32)
        gz = pz.astype(jnp.int32)
        return px, py, pz, gx, gy, gz

    def corner_entries(l, gx, gy, gz):
        scale, res, dense = _LEVELS[l]
        out = []
        for c in range(8):
            cx = gx + (c & 1) if (c & 1) else gx
            cy = gy + ((c >> 1) & 1) if ((c >> 1) & 1) else gy
            cz = gz + ((c >> 2) & 1) if ((c >> 2) & 1) else gz
            if dense:
                idx = cx + cy * res + cz * (res * res)
            else:
                idx = (cx ^ (cy * _P1) ^ (cz * _P2)) & (_T - 1)
            out.append(idx + l * _T)
        return out

    def corner_par4(l, gx, gy, gz):
        scale, res, dense = _LEVELS[l]
        out = []
        for c in range(8):
            cx = gx + (c & 1) if (c & 1) else gx
            cy = gy + ((c >> 1) & 1) if ((c >> 1) & 1) else gy
            cz = gz + ((c >> 2) & 1) if ((c >> 2) & 1) else gz
            if dense and res % 2 == 0:
                p = cx & 1
            else:
                p = (cx ^ cy ^ cz) & 1
            out.append(p * _F)
        return out

    def do_idx(g, b):
        pltpu.sync_copy(pts_hbm.at[pl.ds(base + g * _C, _C)],
                        pts_v.at[pl.ds(b * _C, _C)])

        def sub_idx(s, _):
            x, y, z = load_xyz(s, b)
            for l in range(_N_LEVELS):
                scale, res, dense = _LEVELS[l]
                _, _, _, gx, gy, gz = level_pos(x, y, z, scale)
                es = corner_entries(l, gx, gy, gz)
                for c in range(8):
                    off = s * (_N_LEVELS * 8 * 16) + (l * 8 + c) * 16
                    idx_bufs[b][pl.ds(off, 16)] = es[c] >> 1
            return 0
        lax.fori_loop(0, _SUB, sub_idx, 0)

    def fire(b):
        pltpu.async_copy(table_hbm.at[idx_bufs[b]], rows_bufs[b], sems[b])

    def wait_rows(b):
        pltpu.make_async_copy(table_hbm.at[idx_bufs[b]], rows_bufs[b],
                              sems[b]).wait()

    def do_interp(g, b):
        def sub_interp(s, _):
            x, y, z = load_xyz(s, b)
            for l in range(_N_LEVELS):
                scale, res, dense = _LEVELS[l]
                px, py, pz, gx, gy, gz = level_pos(x, y, z, scale)
                wx = px - gx.astype(jnp.float32)
                wy = py - gy.astype(jnp.float32)
                wz = pz - gz.astype(jnp.float32)
                ux = 1.0 - wx
                uy = 1.0 - wy
                uz = 1.0 - wz
                a = ux * uy
                b2 = wx * uy
                cc = ux * wy
                d = wx * wy
                w8 = [a * uz, b2 * uz, cc * uz, d * uz,
                      a * wz, b2 * wz, cc * wz, d * wz]
                p4 = corner_par4(l, gx, gy, gz)
                acc = [None] * _F
                for c in range(8):
                    off = s * (_N_LEVELS * 8 * 16) + (l * 8 + c) * 16
                    rbase = off + iota
                    for f in range(_F):
                        v = plsc.load_gather(rows_bufs[b], [rbase, p4[c] + f])
                        t = w8[c] * v
                        acc[f] = t if acc[f] is None else acc[f] + t
                row = s * 16 + iota
                for f in range(_F):
                    plsc.store_scatter(
                        feats_v, [row, jnp.full((16,), l * _F + f, jnp.int32)],
                        acc[f])
            return 0
        lax.fori_loop(0, _SUB, sub_interp, 0)
        pltpu.sync_copy(feats_v, feats_hbm.at[pl.ds(base + g * _C, _C)])

    do_idx(0, 0)
    fire(0)

    def pair_body(i, _):
        g0 = i * 2
        do_idx(g0 + 1, 1)
        wait_rows(0)
        fire(1)
        do_interp(g0, 0)

        @pl.when(g0 + 2 < n_chunks)
        def _():
            do_idx(g0 + 2, 0)
        wait_rows(1)

        @pl.when(g0 + 2 < n_chunks)
        def _():
            fire(0)
        do_interp(g0 + 1, 1)
        return 0
    lax.fori_loop(0, n_chunks // 2, pair_body, 0)


_RG = 64


def _relayout_body(src_hbm, dst_hbm, in_v, out_v):
    wid = lax.axis_index("s") * _NC + lax.axis_index("c")
    iota = lax.iota(jnp.int32, 16)
    iota4 = iota * 4
    n = src_hbm.shape[0]
    per_w = n // _NW
    chunk = _RG * 512
    n_chunks = per_w // chunk
    base_w = wid * per_w

    def chunk_body(t, _):
        base = base_w + t * chunk
        pltpu.sync_copy(src_hbm.at[pl.ds(base, chunk)], in_v)

        def g_body(g, _):
            g512 = g * 512
            for f in range(_F):
                for h in range(8):
                    v = in_v[pl.ds(g512 + f * 128 + h * 16, 16)]
                    plsc.store_scatter(out_v, [g512 + h * 64 + f + iota4], v)
            return 0
        lax.fori_loop(0, _RG, g_body, 0)
        pltpu.sync_copy(out_v, dst_hbm.at[pl.ds(base, chunk)])
        return 0
    lax.fori_loop(0, n_chunks, chunk_body, 0)


def _sc_relayout(table1d):
    mesh = plsc.VectorSubcoreMesh(core_axis_name="c", subcore_axis_name="s")
    out = pl.kernel(
        _relayout_body,
        out_type=jax.ShapeDtypeStruct(table1d.shape, jnp.float32),
        mesh=mesh,
        compiler_params=pltpu.CompilerParams(
            needs_layout_passes=False, use_tc_tiling_on_sc=False),
        scratch_types=[
            pltpu.VMEM((_RG * 512,), jnp.float32),
            pltpu.VMEM((_RG * 512,), jnp.float32),
        ],
    )(table1d)
    return out.reshape(_N_LEVELS * _T // 2, 2 * _F)


def _sc_features(pts, table2):
    n_pts = pts.shape[0]
    mesh = plsc.VectorSubcoreMesh(core_axis_name="c", subcore_axis_name="s")
    return pl.kernel(
        _sc_body,
        out_type=jax.ShapeDtypeStruct((n_pts, _N_LEVELS * _F), jnp.float32),
        mesh=mesh,
        compiler_params=pltpu.CompilerParams(
            needs_layout_passes=False, use_tc_tiling_on_sc=False),
        scratch_types=[
            pltpu.VMEM((2 * _C, 3), jnp.float32),
            pltpu.VMEM((_NIDX,), jnp.int32),
            pltpu.VMEM((_NIDX,), jnp.int32),
            pltpu.VMEM((_NIDX, 2 * _F), jnp.float32),
            pltpu.VMEM((_NIDX, 2 * _F), jnp.float32),
            pltpu.VMEM((_C, _N_LEVELS * _F), jnp.float32),
            pltpu.SemaphoreType.DMA,
            pltpu.SemaphoreType.DMA,
        ],
    )(pts, table2)


_SH_C = (0.28209479177387814, -0.48860251190291987, 0.48860251190291987,
         -0.48860251190291987, 1.0925484305920792, -1.0925484305920792,
         0.94617469575755997, -0.31539156525251999, -1.0925484305920792,
         0.54627421529603959, 0.59004358992664352, 2.8906114426405538,
         0.45704579946446572, 0.3731763325901154, 1.4453057213202769)


def _mlp_body(feats_ref, vd_ref, ws1, ws2, wc1g, wc1s, wc2, wc3,
              sig_ref, rgb_ref):
    f = feats_ref[...]
    h1 = jnp.maximum(jnp.dot(f, ws1[...], preferred_element_type=jnp.float32),
                     0.0)
    so = jnp.dot(h1, ws2[...], preferred_element_type=jnp.float32)
    sig_ref[...] = jnp.maximum(so[:, 0:1], 0.0)

    d = vd_ref[...] * 2.0 - 1.0
    x = d[:, 0:1]
    y = d[:, 1:2]
    z = d[:, 2:3]
    xy = x * y
    xz = x * z
    yz = y * z
    x2 = x * x
    y2 = y * y
    z2 = z * z
    sh = jnp.concatenate([
        _SH_C[0] * jnp.ones_like(x),
        _SH_C[1] * y,
        _SH_C[2] * z,
        _SH_C[3] * x,
        _SH_C[4] * xy,
        _SH_C[5] * yz,
        0.94617469575755997 * z2 - 0.31539156525251999,
        _SH_C[8] * xz,
        _SH_C[9] * (x2 - y2),
        _SH_C[10] * y * (-3.0 * x2 + y2),
        _SH_C[11] * xy * z,
        _SH_C[12] * y * (1.0 - 5.0 * z2),
        _SH_C[13] * z * (5.0 * z2 - 3.0),
        _SH_C[12] * x * (1.0 - 5.0 * z2),
        _SH_C[14] * z * (x2 - y2),
        _SH_C[10] * x * (-x2 + 3.0 * y2),
    ], axis=1)

    pre = (jnp.dot(so, wc1g[...], preferred_element_type=jnp.float32)
           + jnp.dot(sh, wc1s[...], preferred_element_type=jnp.float32))
    h = jnp.maximum(pre, 0.0)
    h = jnp.maximum(jnp.dot(h, wc2[...], preferred_element_type=jnp.float32),
                    0.0)
    rgb_ref[...] = jax.nn.sigmoid(
        jnp.dot(h, wc3[...], preferred_element_type=jnp.float32))


def _tc_mlp(feats, viewdirs, W_s1, W_s2, W_c1, W_c2, W_c3):
    n = feats.shape[0]
    B = 2048
    grid = n // B
    wc1g = jnp.concatenate([jnp.zeros((1, 64), jnp.float32), W_c1[:15]], axis=0)
    wc1s = W_c1[15:]
    rep = lambda shape: pl.BlockSpec(shape, lambda i: (0, 0))
    sig, rgb = pl.pallas_call(
        _mlp_body,
        grid=(grid,),
        in_specs=[
            pl.BlockSpec((B, 64), lambda i: (i, 0)),
            pl.BlockSpec((B, 3), lambda i: (i, 0)),
            rep((64, 64)), rep((64, 16)), rep((16, 64)), rep((16, 64)),
            rep((64, 64)), rep((64, 3)),
        ],
        out_specs=[
            pl.BlockSpec((B, 1), lambda i: (i, 0)),
            pl.BlockSpec((B, 3), lambda i: (i, 0)),
        ],
        out_shape=[
            jax.ShapeDtypeStruct((n, 1), jnp.float32),
            jax.ShapeDtypeStruct((n, 3), jnp.float32),
        ],
        compiler_params=pltpu.CompilerParams(
            dimension_semantics=("parallel",)),
    )(feats, viewdirs, W_s1, W_s2, wc1g, wc1s, W_c2, W_c3)
    return sig, rgb


def kernel(pts, viewdirs, table, W_s1, W_s2, W_c1, W_c2, W_c3):
    table1d = table.reshape(
        _N_LEVELS, _T // 128, 128, _F).transpose(0, 1, 3, 2).reshape(-1)
    table2 = _sc_relayout(table1d)
    feats = _sc_features(pts, table2)
    sig, rgb = _tc_mlp(feats, viewdirs, W_s1, W_s2, W_c1, W_c2, W_c3)
    return (sig.reshape(pts.shape[0]), rgb)


# speedup vs baseline: 3.3117x; 1.0005x over previous
import numpy as np
import jax
import jax.numpy as jnp
from jax import lax
from jax.experimental import pallas as pl
from jax.experimental.pallas import tpu as pltpu
from jax.experimental.pallas import tpu_sc as plsc

_N_LEVELS = 16
_F = 4
_T = 2 ** 19
_BASE_RES = 16
_BBOX = 4.0
_P1 = int(np.uint32(2654435761).astype(np.int32))
_P2 = 805459861

_LEVELS = []
for _l in range(_N_LEVELS):
    _scale = float(np.exp2(_l * np.log2(1.5)) * _BASE_RES) - 1.0
    _res = int(np.ceil(_scale)) + 1
    _LEVELS.append((_scale, _res, (_res ** 3) <= _T))

_NC, _NS = 2, 16
_NW = _NC * _NS
_C = 32
_SUB = _C // 16
_NIDX = _C * _N_LEVELS * 8


def _sc_body(pts_hbm, table_hbm, feats_hbm, pts_v, idx_v0, idx_v1,
             rows_v0, rows_v1, feats_v, sem0, sem1):
    n_pts = pts_hbm.shape[0]
    np_per_w = n_pts // _NW
    n_chunks = np_per_w // _C

    wid = lax.axis_index("s") * _NC + lax.axis_index("c")
    base = wid * np_per_w

    iota = lax.iota(jnp.int32, 16)
    col0 = jnp.zeros((16,), jnp.int32)
    col1 = jnp.full((16,), 1, jnp.int32)
    col2 = jnp.full((16,), 2, jnp.int32)
    idx_bufs = (idx_v0, idx_v1)
    rows_bufs = (rows_v0, rows_v1)
    sems = (sem0, sem1)

    def load_xyz(s, b):
        row = b * _C + s * 16 + iota
        x = plsc.load_gather(pts_v, [row, col0])
        y = plsc.load_gather(pts_v, [row, col1])
        z = plsc.load_gather(pts_v, [row, col2])
        x = x * (1.0 / _BBOX) + 0.5
        y = y * (1.0 / _BBOX) + 0.5
        z = z * (1.0 / _BBOX) + 0.5
        return x, y, z

    def level_pos(x, y, z, scale):
        px = x * scale + 0.5
        py = y * scale + 0.5
        pz = z * scale + 0.5
        gx = px.astype(jnp.int32)
        gy = py.astype(jnp.int32)
        gz = pz.astype(jnp.int32)
        return px, py, pz, gx, gy, gz

    def corner_entries(l, gx, gy, gz):
        scale, res, dense = _LEVELS[l]
        out = []
        for c in range(8):
            cx = gx + (c & 1) if (c & 1) else gx
            cy = gy + ((c >> 1) & 1) if ((c >> 1) & 1) else gy
            cz = gz + ((c >> 2) & 1) if ((c >> 2) & 1) else gz
            if dense:
                idx = cx + cy * res + cz * (res * res)
            else:
                idx = (cx ^ (cy * _P1) ^ (cz * _P2)) & (_T - 1)
            out.append(idx + l * _T)
        return out

    def corner_par4(l, gx, gy, gz):
        scale, res, dense = _LEVELS[l]
        out = []
        for c in range(8):
            cx = gx + (c & 1) if (c & 1) else gx
            cy = gy + ((c >> 1) & 1) if ((c >> 1) & 1) else gy
            cz = gz + ((c >> 2) & 1) if ((c >> 2) & 1) else gz
            if dense and res % 2 == 0:
                p = cx & 1
            else:
                p = (cx ^ cy ^ cz) & 1
            out.append(p * _F)
        return out

    def do_idx(g, b):
        pltpu.sync_copy(pts_hbm.at[pl.ds(base + g * _C, _C)],
                        pts_v.at[pl.ds(b * _C, _C)])

        def sub_idx(s, _):
            x, y, z = load_xyz(s, b)
            for l in range(_N_LEVELS):
                scale, res, dense = _LEVELS[l]
                _, _, _, gx, gy, gz = level_pos(x, y, z, scale)
                es = corner_entries(l, gx, gy, gz)
                for c in range(8):
                    off = s * (_N_LEVELS * 8 * 16) + (l * 8 + c) * 16
                    idx_bufs[b][pl.ds(off, 16)] = es[c] >> 1
            return 0
        lax.fori_loop(0, _SUB, sub_idx, 0)

    def fire(b):
        pltpu.async_copy(table_hbm.at[idx_bufs[b]], rows_bufs[b], sems[b])

    def wait_rows(b):
        pltpu.make_async_copy(table_hbm.at[idx_bufs[b]], rows_bufs[b],
                              sems[b]).wait()

    def do_interp(g, b):
        def sub_interp(s, _):
            x, y, z = load_xyz(s, b)
            for l in range(_N_LEVELS):
                scale, res, dense = _LEVELS[l]
                px, py, pz, gx, gy, gz = level_pos(x, y, z, scale)
                wx = px - gx.astype(jnp.float32)
                wy = py - gy.astype(jnp.float32)
                wz = pz - gz.astype(jnp.float32)
                ux = 1.0 - wx
                uy = 1.0 - wy
                uz = 1.0 - wz
                a = ux * uy
                b2 = wx * uy
                cc = ux * wy
                d = wx * wy
                w8 = [a * uz, b2 * uz, cc * uz, d * uz,
                      a * wz, b2 * wz, cc * wz, d * wz]
                p4 = corner_par4(l, gx, gy, gz)
                acc = [None] * _F
                for c in range(8):
                    off = s * (_N_LEVELS * 8 * 16) + (l * 8 + c) * 16
                    rbase = off + iota
                    for f in range(_F):
                        v = plsc.load_gather(rows_bufs[b], [rbase, p4[c] + f])
                        t = w8[c] * v
                        acc[f] = t if acc[f] is None else acc[f] + t
                row = s * 16 + iota
                for f in range(_F):
                    plsc.store_scatter(
                        feats_v, [row, jnp.full((16,), l * _F + f, jnp.int32)],
                        acc[f])
            return 0
        lax.fori_loop(0, _SUB, sub_interp, 0)
        pltpu.sync_copy(feats_v, feats_hbm.at[pl.ds(base + g * _C, _C)])

    do_idx(0, 0)
    fire(0)

    def pair_body(i, _):
        g0 = i * 2
        do_idx(g0 + 1, 1)
        wait_rows(0)
        fire(1)
        do_interp(g0, 0)

        @pl.when(g0 + 2 < n_chunks)
        def _():
            do_idx(g0 + 2, 0)
        wait_rows(1)

        @pl.when(g0 + 2 < n_chunks)
        def _():
            fire(0)
        do_interp(g0 + 1, 1)
        return 0
    lax.fori_loop(0, n_chunks // 2, pair_body, 0)


_RG = 64


def _relayout_body(src_hbm, dst_hbm, in_v, out_v):
    wid = lax.axis_index("s") * _NC + lax.axis_index("c")
    iota = lax.iota(jnp.int32, 16)
    iota4 = iota * 4
    n = src_hbm.shape[0]
    per_w = n // _NW
    chunk = _RG * 512
    n_chunks = per_w // chunk
    base_w = wid * per_w

    def chunk_body(t, _):
        base = base_w + t * chunk
        pltpu.sync_copy(src_hbm.at[pl.ds(base, chunk)], in_v)

        def g_body(g, _):
            g512 = g * 512
            for f in range(_F):
                for h in range(8):
                    v = in_v[pl.ds(g512 + f * 128 + h * 16, 16)]
                    plsc.store_scatter(out_v, [g512 + h * 64 + f + iota4], v)
            return 0
        lax.fori_loop(0, _RG, g_body, 0)
        pltpu.sync_copy(out_v, dst_hbm.at[pl.ds(base, chunk)])
        return 0
    lax.fori_loop(0, n_chunks, chunk_body, 0)


def _sc_relayout(table1d):
    mesh = plsc.VectorSubcoreMesh(core_axis_name="c", subcore_axis_name="s")
    out = pl.kernel(
        _relayout_body,
        out_type=jax.ShapeDtypeStruct(table1d.shape, jnp.float32),
        mesh=mesh,
        compiler_params=pltpu.CompilerParams(
            needs_layout_passes=False, use_tc_tiling_on_sc=False),
        scratch_types=[
            pltpu.VMEM((_RG * 512,), jnp.float32),
            pltpu.VMEM((_RG * 512,), jnp.float32),
        ],
    )(table1d)
    return out.reshape(_N_LEVELS * _T // 2, 2 * _F)


def _sc_features(pts, table2):
    n_pts = pts.shape[0]
    mesh = plsc.VectorSubcoreMesh(core_axis_name="c", subcore_axis_name="s")
    return pl.kernel(
        _sc_body,
        out_type=jax.ShapeDtypeStruct((n_pts, _N_LEVELS * _F), jnp.float32),
        mesh=mesh,
        compiler_params=pltpu.CompilerParams(
            needs_layout_passes=False, use_tc_tiling_on_sc=False),
        scratch_types=[
            pltpu.VMEM((2 * _C, 3), jnp.float32),
            pltpu.VMEM((_NIDX,), jnp.int32),
            pltpu.VMEM((_NIDX,), jnp.int32),
            pltpu.VMEM((_NIDX, 2 * _F), jnp.float32),
            pltpu.VMEM((_NIDX, 2 * _F), jnp.float32),
            pltpu.VMEM((_C, _N_LEVELS * _F), jnp.float32),
            pltpu.SemaphoreType.DMA,
            pltpu.SemaphoreType.DMA,
        ],
    )(pts, table2)


_SH_C = (0.28209479177387814, -0.48860251190291987, 0.48860251190291987,
         -0.48860251190291987, 1.0925484305920792, -1.0925484305920792,
         0.94617469575755997, -0.31539156525251999, -1.0925484305920792,
         0.54627421529603959, 0.59004358992664352, 2.8906114426405538,
         0.45704579946446572, 0.3731763325901154, 1.4453057213202769)


def _mlp_body(feats_ref, vd_ref, ws1, ws2, wc1g, wc1s, wc2, wc3,
              sig_ref, rgb_ref):
    f = feats_ref[...]
    h1 = jnp.maximum(jnp.dot(f, ws1[...], preferred_element_type=jnp.float32),
                     0.0)
    so = jnp.dot(h1, ws2[...], preferred_element_type=jnp.float32)
    sig_ref[...] = jnp.maximum(so[:, 0:1], 0.0)

    d = vd_ref[...] * 2.0 - 1.0
    x = d[:, 0:1]
    y = d[:, 1:2]
    z = d[:, 2:3]
    xy = x * y
    xz = x * z
    yz = y * z
    x2 = x * x
    y2 = y * y
    z2 = z * z
    sh = jnp.concatenate([
        _SH_C[0] * jnp.ones_like(x),
        _SH_C[1] * y,
        _SH_C[2] * z,
        _SH_C[3] * x,
        _SH_C[4] * xy,
        _SH_C[5] * yz,
        0.94617469575755997 * z2 - 0.31539156525251999,
        _SH_C[8] * xz,
        _SH_C[9] * (x2 - y2),
        _SH_C[10] * y * (-3.0 * x2 + y2),
        _SH_C[11] * xy * z,
        _SH_C[12] * y * (1.0 - 5.0 * z2),
        _SH_C[13] * z * (5.0 * z2 - 3.0),
        _SH_C[12] * x * (1.0 - 5.0 * z2),
        _SH_C[14] * z * (x2 - y2),
        _SH_C[10] * x * (-x2 + 3.0 * y2),
    ], axis=1)

    pre = (jnp.dot(so, wc1g[...], preferred_element_type=jnp.float32)
           + jnp.dot(sh, wc1s[...], preferred_element_type=jnp.float32))
    h = jnp.maximum(pre, 0.0)
    h = jnp.maximum(jnp.dot(h, wc2[...], preferred_element_type=jnp.float32),
                    0.0)
    rgb_ref[...] = jax.nn.sigmoid(
        jnp.dot(h, wc3[...], preferred_element_type=jnp.float32))


def _tc_mlp(feats, viewdirs, W_s1, W_s2, W_c1, W_c2, W_c3):
    n = feats.shape[0]
    B = 2048
    grid = n // B
    wc1g = jnp.concatenate([jnp.zeros((1, 64), jnp.float32), W_c1[:15]], axis=0)
    wc1s = W_c1[15:]
    rep = lambda shape: pl.BlockSpec(shape, lambda i: (0, 0))
    sig, rgb = pl.pallas_call(
        _mlp_body,
        grid=(grid,),
        in_specs=[
            pl.BlockSpec((B, 64), lambda i: (i, 0)),
            pl.BlockSpec((B, 3), lambda i: (i, 0)),
            rep((64, 64)), rep((64, 16)), rep((16, 64)), rep((16, 64)),
            rep((64, 64)), rep((64, 3)),
        ],
        out_specs=[
            pl.BlockSpec((B, 1), lambda i: (i, 0)),
            pl.BlockSpec((B, 3), lambda i: (i, 0)),
        ],
        out_shape=[
            jax.ShapeDtypeStruct((n, 1), jnp.float32),
            jax.ShapeDtypeStruct((n, 3), jnp.float32),
        ],
        compiler_params=pltpu.CompilerParams(
            dimension_semantics=("parallel",)),
    )(feats, viewdirs, W_s1, W_s2, wc1g, wc1s, W_c2, W_c3)
    return sig, rgb


def kernel(pts, viewdirs, table, W_s1, W_s2, W_c1, W_c2, W_c3):
    table1d = table.reshape(
        _N_LEVELS, _T // 128, 128, _F).transpose(0, 1, 3, 2).reshape(-1)
    table2 = _sc_relayout(table1d)
    feats = _sc_features(pts, table2)
    sig, rgb = _tc_mlp(feats, viewdirs, W_s1, W_s2, W_c1, W_c2, W_c3)
    return (sig.reshape(pts.shape[0]), rgb)
